# Initial kernel scaffold; baseline (speedup 1.0000x reference)
import jax
import jax.numpy as jnp
from jax.experimental import pallas as pl


def kernel(inputs, token_table, pos_table):
    raise NotImplementedError("write your pallas kernel here")


# speedup vs baseline: 1.4609x; 1.4609x over previous
import functools

import jax
import jax.numpy as jnp
from jax import lax
from jax.experimental import pallas as pl
from jax.experimental.pallas import tpu as pltpu
from jax.experimental.pallas import tpu_sc as plsc

VOCAB = 1000000
SEQ_LEN = 200
EMBED = 32
BATCH = 4096

NC, NS = 2, 16
NW = NC * NS
TOTAL = BATCH * SEQ_LEN
PER_W = TOTAL // NW
SEQ_PER_CHUNK = 4
CHUNK = SEQ_PER_CHUNK * SEQ_LEN
NCHUNK = PER_W // CHUNK
GATHER_W = 100
NGATHER = CHUNK // GATHER_W
IDX_ROWS = TOTAL // GATHER_W
NBUF = 4


def _body(idx_hbm, table_hbm, pos_hbm, out_hbm, pos_v,
          idx0, idx1, idx2, idx3, row0, row1, row2, row3,
          g0, g1, g2, g3, w0, w1, w2, w3):
  idxs = [idx0, idx1, idx2, idx3]
  rows = [row0, row1, row2, row3]
  gsem = [g0, g1, g2, g3]
  wsem = [w0, w1, w2, w3]

  wid = lax.axis_index("s") * NC + lax.axis_index("c")
  wbase = wid * PER_W
  irow = wid * (PER_W // GATHER_W)

  pltpu.sync_copy(pos_hbm, pos_v)

  def start_chunk(g, b):
    pltpu.sync_copy(idx_hbm.at[pl.ds(irow + g * NGATHER, NGATHER)], idxs[b])
    descs = []
    for j in range(NGATHER):
      descs.append(pltpu.async_copy(
          table_hbm.at[idxs[b].at[j]],
          rows[b].at[pl.ds(j * GATHER_W, GATHER_W)],
          gsem[b]))
    return descs

  gdesc = [None] * NBUF
  wdesc = [None] * NBUF
  for b in range(NBUF):
    gdesc[b] = start_chunk(b, b)

  for g in range(NCHUNK):
    b = g % NBUF
    for d in gdesc[b]:
      d.wait()

    def add_pos(l, carry, rbuf=rows[b]):
      p0 = pos_v[l, pl.ds(0, 16)]
      p1 = pos_v[l, pl.ds(16, 16)]
      for s in range(SEQ_PER_CHUNK):
        r = s * SEQ_LEN + l
        plsc.addupdate(rbuf.at[r, pl.ds(0, 16)], p0)
        plsc.addupdate(rbuf.at[r, pl.ds(16, 16)], p1)
      return carry
    lax.fori_loop(0, SEQ_LEN, add_pos, 0)

    wdesc[b] = pltpu.async_copy(
        rows[b], out_hbm.at[pl.ds(wbase + g * CHUNK, CHUNK)], wsem[b])
    if g + NBUF < NCHUNK:
      wdesc[b].wait()
      wdesc[b] = None
      gdesc[b] = start_chunk(g + NBUF, b)

  for b in range(NBUF):
    if wdesc[b] is not None:
      wdesc[b].wait()


@functools.partial(jax.jit, donate_argnums=())
def _run(idx2d, token_table, pos_table):
  mesh = plsc.VectorSubcoreMesh(core_axis_name="c", subcore_axis_name="s")
  scratch = (
      [pltpu.VMEM((SEQ_LEN, EMBED), jnp.float32)]
      + [pltpu.VMEM((NGATHER, GATHER_W), jnp.int32) for _ in range(NBUF)]
      + [pltpu.VMEM((CHUNK, EMBED), jnp.float32) for _ in range(NBUF)]
      + [pltpu.SemaphoreType.DMA for _ in range(2 * NBUF)]
  )
  return pl.kernel(
      _body,
      out_type=jax.ShapeDtypeStruct((TOTAL, EMBED), jnp.float32),
      mesh=mesh,
      scratch_types=scratch,
      compiler_params=pltpu.CompilerParams(use_tc_tiling_on_sc=False),
  )(idx2d, token_table, pos_table)


def kernel(inputs, token_table, pos_table):
  idx2d = inputs.reshape(IDX_ROWS, GATHER_W)
  out = _run(idx2d, token_table, pos_table)
  return out.reshape(BATCH, SEQ_LEN, EMBED)


# speedup vs baseline: 1.4912x; 1.0207x over previous
import functools

import jax
import jax.numpy as jnp
from jax import lax
from jax.experimental import pallas as pl
from jax.experimental.pallas import tpu as pltpu
from jax.experimental.pallas import tpu_sc as plsc

VOCAB = 1000000
SEQ_LEN = 200
EMBED = 32
BATCH = 4096

NC, NS = 2, 16
NW = NC * NS
SEQ_PER_W = BATCH // NW
SEQ_PER_CHUNK = 4
CHUNK = SEQ_PER_CHUNK * SEQ_LEN
NCHUNK = SEQ_PER_W // SEQ_PER_CHUNK
SPLITS = ((0, 128), (128, 72))
NBUF = 3


def _body(inp_hbm, table_hbm, pos_hbm, out_hbm, idx_v, pos_v,
          row0, row1, row2, isem, g0, g1, g2, w0, w1, w2):
  rows = [row0, row1, row2]
  gsem = [g0, g1, g2]
  wsem = [w0, w1, w2]

  wid = lax.axis_index("s") * NC + lax.axis_index("c")
  wseq = wid * SEQ_PER_W

  pltpu.async_copy(inp_hbm.at[pl.ds(wseq, SEQ_PER_W)], idx_v, isem).wait()
  pltpu.sync_copy(pos_hbm, pos_v)

  def start_chunk(g, b):
    descs = []
    for s in range(SEQ_PER_CHUNK):
      for off, width in SPLITS:
        descs.append(pltpu.async_copy(
            table_hbm.at[idx_v.at[g * SEQ_PER_CHUNK + s, pl.ds(off, width)]],
            rows[b].at[s, pl.ds(off, width)],
            gsem[b]))
    return descs

  gdesc = [None] * NBUF
  wdesc = [None] * NBUF
  for b in range(NBUF - 1):
    gdesc[b] = start_chunk(b, b)

  for g in range(NCHUNK):
    b = g % NBUF
    for d in gdesc[b]:
      d.wait()

    def add_pos(l, carry, rbuf=rows[b]):
      p0 = pos_v[l, pl.ds(0, 16)]
      p1 = pos_v[l, pl.ds(16, 16)]
      for s in range(SEQ_PER_CHUNK):
        plsc.addupdate(rbuf.at[s, l, pl.ds(0, 16)], p0)
        plsc.addupdate(rbuf.at[s, l, pl.ds(16, 16)], p1)
      return carry
    lax.fori_loop(0, SEQ_LEN, add_pos, 0)

    wdesc[b] = pltpu.async_copy(
        rows[b], out_hbm.at[pl.ds(wseq + g * SEQ_PER_CHUNK, SEQ_PER_CHUNK)],
        wsem[b])

    if g + NBUF - 1 < NCHUNK:
      nb = (g + NBUF - 1) % NBUF
      if wdesc[nb] is not None:
        wdesc[nb].wait()
        wdesc[nb] = None
      gdesc[nb] = start_chunk(g + NBUF - 1, nb)

  for b in range(NBUF):
    if wdesc[b] is not None:
      wdesc[b].wait()


@jax.jit
def _run(inputs, token_table, pos_table):
  mesh = plsc.VectorSubcoreMesh(core_axis_name="c", subcore_axis_name="s")
  scratch = (
      [pltpu.VMEM((SEQ_PER_W, SEQ_LEN), jnp.int32),
       pltpu.VMEM((SEQ_LEN, EMBED), jnp.float32)]
      + [pltpu.VMEM((SEQ_PER_CHUNK, SEQ_LEN, EMBED), jnp.float32)
         for _ in range(NBUF)]
      + [pltpu.SemaphoreType.DMA for _ in range(2 * NBUF + 1)]
  )
  return pl.kernel(
      _body,
      out_type=jax.ShapeDtypeStruct((BATCH, SEQ_LEN, EMBED), jnp.float32),
      mesh=mesh,
      scratch_types=scratch,
      compiler_params=pltpu.CompilerParams(use_tc_tiling_on_sc=False),
  )(inputs, token_table, pos_table)


def kernel(inputs, token_table, pos_table):
  return _run(inputs, token_table, pos_table)
